# baseline (speedup 1.0000x reference)
import functools

import jax
import jax.numpy as jnp
from jax.experimental import pallas as pl
from jax.experimental.pallas import tpu as pltpu

_B, _P, _T, _C = 32, 24564, 50, 21
_L = 128
_R = (_P + _L - 1) // _L
_PP = _R * _L
_THRESH = 0.5
_NEG_POS = 3


def _loss_kernel(gt_ref, loc_ref, score_ref, pri_ref, o_l, o_c, o_k, o_n):
    f32 = jnp.float32
    row = jax.lax.broadcasted_iota(jnp.int32, (_R, _L), 0)
    col = jax.lax.broadcasted_iota(jnp.int32, (_R, _L), 1)
    pidx = row * _L + col
    valid = pidx < _P

    cx = pri_ref[0]
    cy = pri_ref[1]
    pw = pri_ref[2]
    ph = pri_ref[3]
    px1 = cx - pw / 2.0
    py1 = cy - ph / 2.0
    px2 = cx + pw / 2.0
    py2 = cy + ph / 2.0
    area_p = (px2 - px1) * (py2 - py1)

    def match_body(t, carry):
        bo, bi, fidx, fmask = carry
        tx1 = gt_ref[0, t, 0]
        ty1 = gt_ref[0, t, 1]
        tx2 = gt_ref[0, t, 2]
        ty2 = gt_ref[0, t, 3]
        iw = jnp.maximum(jnp.minimum(tx2, px2) - jnp.maximum(tx1, px1), 0.0)
        ih = jnp.maximum(jnp.minimum(ty2, py2) - jnp.maximum(ty1, py1), 0.0)
        inter = iw * ih
        area_t = (tx2 - tx1) * (ty2 - ty1)
        ov = inter / (area_t + area_p - inter)
        ov = jnp.where(valid, ov, -1.0)
        upd = ov > bo
        bo = jnp.where(upd, ov, bo)
        bi = jnp.where(upd, t, bi)
        m = jnp.max(ov)
        bp = jnp.min(jnp.where(ov == m, pidx, _PP))
        hit = pidx == bp
        fidx = jnp.where(hit, t, fidx)
        fmask = jnp.where(hit, 1, fmask)
        return bo, bi, fidx, fmask

    init = (
        jnp.full((_R, _L), -jnp.inf, f32),
        jnp.zeros((_R, _L), jnp.int32),
        jnp.zeros((_R, _L), jnp.int32),
        jnp.zeros((_R, _L), jnp.int32),
    )
    bo, bi, fidx, fmask = jax.lax.fori_loop(0, _T, match_body, init)
    forced = fmask > 0
    bo = jnp.where(forced, 2.0, bo)
    bi = jnp.where(forced, fidx, bi)

    def gather_body(t, carry):
        lab, m1, m2, m3, m4 = carry
        sel = bi == t
        lab = jnp.where(sel, gt_ref[0, t, 4], lab)
        m1 = jnp.where(sel, gt_ref[0, t, 0], m1)
        m2 = jnp.where(sel, gt_ref[0, t, 1], m2)
        m3 = jnp.where(sel, gt_ref[0, t, 2], m3)
        m4 = jnp.where(sel, gt_ref[0, t, 3], m4)
        return lab, m1, m2, m3, m4

    z = jnp.zeros((_R, _L), f32)
    lab, mx1, my1, mx2, my2 = jax.lax.fori_loop(
        0, _T, gather_body, (z, z, z, z, z))

    conf = jnp.where(bo < _THRESH, 0.0, lab + 1.0)
    conf = jnp.where(valid, conf, 0.0)
    pos = conf > 0.0

    g0 = ((mx1 + mx2) / 2.0 - cx) / (0.1 * pw)
    g1 = ((my1 + my2) / 2.0 - cy) / (0.1 * ph)
    g2 = jnp.log((mx2 - mx1) / pw) / 0.2
    g3 = jnp.log((my2 - my1) / ph) / 0.2

    def sl1(x):
        ax = jnp.abs(x)
        return jnp.where(ax < 1.0, 0.5 * x * x, ax - 0.5)

    lsum = (sl1(loc_ref[0, 0] - g0) + sl1(loc_ref[0, 1] - g1)
            + sl1(loc_ref[0, 2] - g2) + sl1(loc_ref[0, 3] - g3))
    loss_l = jnp.sum(jnp.where(pos, lsum, 0.0))

    m = score_ref[0, 0]
    for c in range(1, _C):
        m = jnp.maximum(m, score_ref[0, c])
    s = jnp.zeros((_R, _L), f32)
    pk = jnp.zeros((_R, _L), f32)
    for c in range(_C):
        x = score_ref[0, c]
        s = s + jnp.exp(x - m)
        pk = jnp.where(conf == c, x, pk)
    lse = m + jnp.log(s)
    ce = lse - pk

    pos_ce = jnp.sum(jnp.where(pos, ce, 0.0))
    num_pos = jnp.sum(pos.astype(jnp.int32))
    k = jnp.minimum(_NEG_POS * num_pos, _P - 1)

    loss_c = jnp.where(pos, 0.0, ce)
    loss_c = jnp.where(valid, loss_c, 0.0)
    u = jax.lax.bitcast_convert_type(loss_c, jnp.int32)

    def radix_body(i, t_acc):
        bit = 30 - i
        cand = t_acc | jax.lax.shift_left(jnp.int32(1), bit)
        cnt = jnp.sum((u >= cand).astype(jnp.int32))
        return jnp.where(cnt >= k, cand, t_acc)

    t_fin = jax.lax.fori_loop(0, 31, radix_body, jnp.int32(0))
    tau = jax.lax.bitcast_convert_type(t_fin, f32)
    gt_mask = loss_c > tau
    sum_gt = jnp.sum(jnp.where(gt_mask, loss_c, 0.0))
    cnt_gt = jnp.sum(gt_mask.astype(jnp.int32))
    topk = sum_gt + (k - cnt_gt).astype(f32) * tau
    topk = jnp.where(k > 0, topk, 0.0)

    o_l[...] = jnp.full((1, 1, _L), loss_l, f32)
    o_c[...] = jnp.full((1, 1, _L), pos_ce, f32)
    o_k[...] = jnp.full((1, 1, _L), topk, f32)
    o_n[...] = jnp.full((1, 1, _L), num_pos.astype(f32), f32)


@jax.jit
def kernel(pred_loc, pred_score, priors, gt_data):
    f32 = jnp.float32
    pad = _PP - _P
    loc_t = jnp.pad(pred_loc.transpose(0, 2, 1), ((0, 0), (0, 0), (0, pad)))
    loc_t = loc_t.reshape(_B, 4, _R, _L)
    score_t = jnp.pad(pred_score.transpose(0, 2, 1),
                      ((0, 0), (0, 0), (0, pad)))
    score_t = score_t.reshape(_B, _C, _R, _L)
    pad_rows = jnp.tile(jnp.array([[0.5, 0.5, 1.0, 1.0]], f32), (pad, 1))
    pri_t = jnp.concatenate([priors, pad_rows], axis=0).T.reshape(4, _R, _L)

    grid = (_B,)
    out_shape = [jax.ShapeDtypeStruct((_B, 1, _L), f32)] * 4
    res = pl.pallas_call(
        _loss_kernel,
        grid=grid,
        in_specs=[
            pl.BlockSpec((1, _T, 5), lambda b: (b, 0, 0),
                         memory_space=pltpu.SMEM),
            pl.BlockSpec((1, 4, _R, _L), lambda b: (b, 0, 0, 0)),
            pl.BlockSpec((1, _C, _R, _L), lambda b: (b, 0, 0, 0)),
            pl.BlockSpec((4, _R, _L), lambda b: (0, 0, 0)),
        ],
        out_specs=[pl.BlockSpec((1, 1, _L), lambda b: (b, 0, 0))] * 4,
        out_shape=out_shape,
        compiler_params=pltpu.CompilerParams(
            dimension_semantics=("parallel",)),
    )(gt_data, loc_t, score_t, pri_t)
    l, c, k, n = (r[:, 0, 0] for r in res)
    n_tot = jnp.sum(n)
    return jnp.sum(l) / n_tot, (jnp.sum(c) + jnp.sum(k)) / n_tot


# speedup vs baseline: 1.2163x; 1.2163x over previous
import jax
import jax.numpy as jnp
from jax.experimental import pallas as pl
from jax.experimental.pallas import tpu as pltpu

_B, _P, _T, _C = 32, 24564, 50, 21
_L = 128
_R = (_P + _L - 1) // _L
_PP = _R * _L
_THRESH = 0.5
_NEG_POS = 3
_IMGS = 2


def _loss_kernel(gt_ref, loc_ref, score_ref, pri_ref, o_l, o_c, o_k, o_n):
    f32 = jnp.float32
    row = jax.lax.broadcasted_iota(jnp.int32, (_R, _L), 0)
    col = jax.lax.broadcasted_iota(jnp.int32, (_R, _L), 1)
    pidx = row * _L + col
    valid = pidx < _P

    cx = pri_ref[0]
    cy = pri_ref[1]
    pw = pri_ref[2]
    ph = pri_ref[3]
    px1 = cx - pw / 2.0
    py1 = cy - ph / 2.0
    px2 = cx + pw / 2.0
    py2 = cy + ph / 2.0
    area_p = (px2 - px1) * (py2 - py1)

    def match_one(i, t, bo, bi, fidx, fmask):
        tx1 = gt_ref[i, t, 0]
        ty1 = gt_ref[i, t, 1]
        tx2 = gt_ref[i, t, 2]
        ty2 = gt_ref[i, t, 3]
        iw = jnp.maximum(jnp.minimum(tx2, px2) - jnp.maximum(tx1, px1), 0.0)
        ih = jnp.maximum(jnp.minimum(ty2, py2) - jnp.maximum(ty1, py1), 0.0)
        inter = iw * ih
        area_t = (tx2 - tx1) * (ty2 - ty1)
        ov = inter / (area_t + area_p - inter)
        ov = jnp.where(valid, ov, -1.0)
        upd = ov > bo
        bo = jnp.where(upd, ov, bo)
        bi = jnp.where(upd, t, bi)
        m = jnp.max(ov)
        bp = jnp.min(jnp.where(ov == m, pidx, _PP))
        hit = pidx == bp
        fidx = jnp.where(hit, t, fidx)
        fmask = jnp.where(hit, 1, fmask)
        return bo, bi, fidx, fmask

    def match_body(t, carry):
        return tuple(match_one(i, t, *carry[i]) for i in range(_IMGS))

    one_init = (
        jnp.full((_R, _L), -jnp.inf, f32),
        jnp.zeros((_R, _L), jnp.int32),
        jnp.zeros((_R, _L), jnp.int32),
        jnp.zeros((_R, _L), jnp.int32),
    )
    mres = jax.lax.fori_loop(0, _T, match_body, (one_init,) * _IMGS)

    bis = []
    for i in range(_IMGS):
        bo, bi, fidx, fmask = mres[i]
        forced = fmask > 0
        bis.append((jnp.where(forced, 2.0, bo), jnp.where(forced, fidx, bi)))

    def gather_one(i, t, bi, lab, m1, m2, m3, m4):
        sel = bi == t
        lab = jnp.where(sel, gt_ref[i, t, 4], lab)
        m1 = jnp.where(sel, gt_ref[i, t, 0], m1)
        m2 = jnp.where(sel, gt_ref[i, t, 1], m2)
        m3 = jnp.where(sel, gt_ref[i, t, 2], m3)
        m4 = jnp.where(sel, gt_ref[i, t, 3], m4)
        return lab, m1, m2, m3, m4

    def gather_body(t, carry):
        return tuple(gather_one(i, t, bis[i][1], *carry[i])
                     for i in range(_IMGS))

    z = jnp.zeros((_R, _L), f32)
    gres = jax.lax.fori_loop(0, _T, gather_body, ((z,) * 5,) * _IMGS)

    def sl1(x):
        ax = jnp.abs(x)
        return jnp.where(ax < 1.0, 0.5 * x * x, ax - 0.5)

    confs, loss_ls = [], []
    for i in range(_IMGS):
        bo, _ = bis[i]
        lab, mx1, my1, mx2, my2 = gres[i]
        conf = jnp.where(bo < _THRESH, 0.0, lab + 1.0)
        conf = jnp.where(valid, conf, 0.0)
        confs.append(conf)
        g0 = ((mx1 + mx2) / 2.0 - cx) / (0.1 * pw)
        g1 = ((my1 + my2) / 2.0 - cy) / (0.1 * ph)
        g2 = jnp.log((mx2 - mx1) / pw) / 0.2
        g3 = jnp.log((my2 - my1) / ph) / 0.2
        lsum = (sl1(loc_ref[i, 0] - g0) + sl1(loc_ref[i, 1] - g1)
                + sl1(loc_ref[i, 2] - g2) + sl1(loc_ref[i, 3] - g3))
        loss_ls.append(jnp.sum(jnp.where(conf > 0.0, lsum, 0.0)))

    ms = [score_ref[i, 0] for i in range(_IMGS)]
    for c in range(1, _C):
        ms = [jnp.maximum(ms[i], score_ref[i, c]) for i in range(_IMGS)]
    ss = [jnp.zeros((_R, _L), f32) for _ in range(_IMGS)]
    pks = [jnp.zeros((_R, _L), f32) for _ in range(_IMGS)]
    for c in range(_C):
        for i in range(_IMGS):
            x = score_ref[i, c]
            ss[i] = ss[i] + jnp.exp(x - ms[i])
            pks[i] = jnp.where(confs[i] == c, x, pks[i])

    ces, loss_cs, us, pos_ces, num_poss, ks = [], [], [], [], [], []
    for i in range(_IMGS):
        ce = ms[i] + jnp.log(ss[i]) - pks[i]
        pos = confs[i] > 0.0
        pos_ces.append(jnp.sum(jnp.where(pos, ce, 0.0)))
        num_pos = jnp.sum(pos.astype(jnp.int32))
        num_poss.append(num_pos)
        ks.append(jnp.minimum(_NEG_POS * num_pos, _P - 1))
        loss_c = jnp.where(pos, 0.0, ce)
        loss_c = jnp.where(valid, loss_c, 0.0)
        loss_cs.append(loss_c)
        us.append(jax.lax.bitcast_convert_type(loss_c, jnp.int32))

    def radix_body(r, accs):
        out = []
        for i in range(_IMGS):
            t_acc = accs[i]
            hi = jax.lax.shift_left(jnp.int32(1), 30 - 2 * r)
            lo = jax.lax.shift_left(jnp.int32(1), 29 - 2 * r)
            c1 = t_acc | lo
            c2 = t_acc | hi
            c3 = t_acc | hi | lo
            n1 = jnp.sum((us[i] >= c1).astype(jnp.int32))
            n2 = jnp.sum((us[i] >= c2).astype(jnp.int32))
            n3 = jnp.sum((us[i] >= c3).astype(jnp.int32))
            k = ks[i]
            t_acc = jnp.where(
                n3 >= k, c3,
                jnp.where(n2 >= k, c2, jnp.where(n1 >= k, c1, t_acc)))
            out.append(t_acc)
        return tuple(out)

    t_fins = jax.lax.fori_loop(0, 15, radix_body,
                               (jnp.int32(0),) * _IMGS)

    def last_body(accs):
        out = []
        for i in range(_IMGS):
            cand = accs[i] | 1
            cnt = jnp.sum((us[i] >= cand).astype(jnp.int32))
            out.append(jnp.where(cnt >= ks[i], cand, accs[i]))
        return tuple(out)

    t_fins = last_body(t_fins)

    for i in range(_IMGS):
        tau = jax.lax.bitcast_convert_type(t_fins[i], f32)
        loss_c = loss_cs[i]
        gt_mask = loss_c > tau
        sum_gt = jnp.sum(jnp.where(gt_mask, loss_c, 0.0))
        cnt_gt = jnp.sum(gt_mask.astype(jnp.int32))
        k = ks[i]
        topk = sum_gt + (k - cnt_gt).astype(f32) * tau
        topk = jnp.where(k > 0, topk, 0.0)
        o_l[i, :, :] = jnp.full((1, _L), loss_ls[i], f32)
        o_c[i, :, :] = jnp.full((1, _L), pos_ces[i], f32)
        o_k[i, :, :] = jnp.full((1, _L), topk, f32)
        o_n[i, :, :] = jnp.full((1, _L), num_poss[i].astype(f32), f32)


@jax.jit
def kernel(pred_loc, pred_score, priors, gt_data):
    f32 = jnp.float32
    pad = _PP - _P
    loc_t = jnp.pad(pred_loc.transpose(0, 2, 1), ((0, 0), (0, 0), (0, pad)))
    loc_t = loc_t.reshape(_B, 4, _R, _L)
    score_t = jnp.pad(pred_score.transpose(0, 2, 1),
                      ((0, 0), (0, 0), (0, pad)))
    score_t = score_t.reshape(_B, _C, _R, _L)
    pad_rows = jnp.tile(jnp.array([[0.5, 0.5, 1.0, 1.0]], f32), (pad, 1))
    pri_t = jnp.concatenate([priors, pad_rows], axis=0).T.reshape(4, _R, _L)

    grid = (_B // _IMGS,)
    out_shape = [jax.ShapeDtypeStruct((_B, 1, _L), f32)] * 4
    res = pl.pallas_call(
        _loss_kernel,
        grid=grid,
        in_specs=[
            pl.BlockSpec((_IMGS, _T, 5), lambda b: (b, 0, 0),
                         memory_space=pltpu.SMEM),
            pl.BlockSpec((_IMGS, 4, _R, _L), lambda b: (b, 0, 0, 0)),
            pl.BlockSpec((_IMGS, _C, _R, _L), lambda b: (b, 0, 0, 0)),
            pl.BlockSpec((4, _R, _L), lambda b: (0, 0, 0)),
        ],
        out_specs=[pl.BlockSpec((_IMGS, 1, _L), lambda b: (b, 0, 0))] * 4,
        out_shape=out_shape,
        compiler_params=pltpu.CompilerParams(
            dimension_semantics=("parallel",),
            vmem_limit_bytes=100 * 1024 * 1024),
    )(gt_data, loc_t, score_t, pri_t)
    l, c, k, n = (r[:, 0, 0] for r in res)
    n_tot = jnp.sum(n)
    return jnp.sum(l) / n_tot, (jnp.sum(c) + jnp.sum(k)) / n_tot


# speedup vs baseline: 1.3145x; 1.0808x over previous
import jax
import jax.numpy as jnp
from jax.experimental import pallas as pl
from jax.experimental.pallas import tpu as pltpu

_B, _P, _T, _C = 32, 24564, 50, 21
_L = 128
_R = (_P + _L - 1) // _L
_PP = _R * _L
_THRESH = 0.5
_NEG_POS = 3
_IMGS = 4


def _loss_kernel(gt_ref, loc_ref, score_ref, pri_ref, o_l, o_c, o_k, o_n):
    f32 = jnp.float32
    row = jax.lax.broadcasted_iota(jnp.int32, (_R, _L), 0)
    col = jax.lax.broadcasted_iota(jnp.int32, (_R, _L), 1)
    pidx = row * _L + col
    valid = pidx < _P

    cx = pri_ref[0]
    cy = pri_ref[1]
    pw = pri_ref[2]
    ph = pri_ref[3]
    px1 = cx - pw / 2.0
    py1 = cy - ph / 2.0
    px2 = cx + pw / 2.0
    py2 = cy + ph / 2.0
    area_p = (px2 - px1) * (py2 - py1)

    def match_one(i, t, bo, bi, fidx, fmask):
        tx1 = gt_ref[i, t, 0]
        ty1 = gt_ref[i, t, 1]
        tx2 = gt_ref[i, t, 2]
        ty2 = gt_ref[i, t, 3]
        iw = jnp.maximum(jnp.minimum(tx2, px2) - jnp.maximum(tx1, px1), 0.0)
        ih = jnp.maximum(jnp.minimum(ty2, py2) - jnp.maximum(ty1, py1), 0.0)
        inter = iw * ih
        area_t = (tx2 - tx1) * (ty2 - ty1)
        ov = inter / (area_t + area_p - inter)
        ov = jnp.where(valid, ov, -1.0)
        upd = ov > bo
        bo = jnp.where(upd, ov, bo)
        bi = jnp.where(upd, t, bi)
        m = jnp.max(ov)
        bp = jnp.min(jnp.where(ov == m, pidx, _PP))
        hit = pidx == bp
        fidx = jnp.where(hit, t, fidx)
        fmask = jnp.where(hit, 1, fmask)
        return bo, bi, fidx, fmask

    def match_body(t, carry):
        return tuple(match_one(i, t, *carry[i]) for i in range(_IMGS))

    one_init = (
        jnp.full((_R, _L), -jnp.inf, f32),
        jnp.zeros((_R, _L), jnp.int32),
        jnp.zeros((_R, _L), jnp.int32),
        jnp.zeros((_R, _L), jnp.int32),
    )
    mres = jax.lax.fori_loop(0, _T, match_body, (one_init,) * _IMGS)

    bis = []
    for i in range(_IMGS):
        bo, bi, fidx, fmask = mres[i]
        forced = fmask > 0
        bis.append((jnp.where(forced, 2.0, bo), jnp.where(forced, fidx, bi)))

    def gather_one(i, t, bi, lab, m1, m2, m3, m4):
        sel = bi == t
        lab = jnp.where(sel, gt_ref[i, t, 4], lab)
        m1 = jnp.where(sel, gt_ref[i, t, 0], m1)
        m2 = jnp.where(sel, gt_ref[i, t, 1], m2)
        m3 = jnp.where(sel, gt_ref[i, t, 2], m3)
        m4 = jnp.where(sel, gt_ref[i, t, 3], m4)
        return lab, m1, m2, m3, m4

    def gather_body(t, carry):
        return tuple(gather_one(i, t, bis[i][1], *carry[i])
                     for i in range(_IMGS))

    z = jnp.zeros((_R, _L), f32)
    gres = jax.lax.fori_loop(0, _T, gather_body, ((z,) * 5,) * _IMGS)

    def sl1(x):
        ax = jnp.abs(x)
        return jnp.where(ax < 1.0, 0.5 * x * x, ax - 0.5)

    confs, loss_ls = [], []
    for i in range(_IMGS):
        bo, _ = bis[i]
        lab, mx1, my1, mx2, my2 = gres[i]
        conf = jnp.where(bo < _THRESH, 0.0, lab + 1.0)
        conf = jnp.where(valid, conf, 0.0)
        confs.append(conf)
        g0 = ((mx1 + mx2) / 2.0 - cx) / (0.1 * pw)
        g1 = ((my1 + my2) / 2.0 - cy) / (0.1 * ph)
        g2 = jnp.log((mx2 - mx1) / pw) / 0.2
        g3 = jnp.log((my2 - my1) / ph) / 0.2
        lsum = (sl1(loc_ref[i, 0] - g0) + sl1(loc_ref[i, 1] - g1)
                + sl1(loc_ref[i, 2] - g2) + sl1(loc_ref[i, 3] - g3))
        loss_ls.append(jnp.sum(jnp.where(conf > 0.0, lsum, 0.0)))

    ms = [score_ref[i, 0] for i in range(_IMGS)]
    for c in range(1, _C):
        ms = [jnp.maximum(ms[i], score_ref[i, c]) for i in range(_IMGS)]
    ss = [jnp.zeros((_R, _L), f32) for _ in range(_IMGS)]
    pks = [jnp.zeros((_R, _L), f32) for _ in range(_IMGS)]
    for c in range(_C):
        for i in range(_IMGS):
            x = score_ref[i, c]
            ss[i] = ss[i] + jnp.exp(x - ms[i])
            pks[i] = jnp.where(confs[i] == c, x, pks[i])

    ces, loss_cs, us, pos_ces, num_poss, ks = [], [], [], [], [], []
    for i in range(_IMGS):
        ce = ms[i] + jnp.log(ss[i]) - pks[i]
        pos = confs[i] > 0.0
        pos_ces.append(jnp.sum(jnp.where(pos, ce, 0.0)))
        num_pos = jnp.sum(pos.astype(jnp.int32))
        num_poss.append(num_pos)
        ks.append(jnp.minimum(_NEG_POS * num_pos, _P - 1))
        loss_c = jnp.where(pos, 0.0, ce)
        loss_c = jnp.where(valid, loss_c, 0.0)
        loss_cs.append(loss_c)
        us.append(jax.lax.bitcast_convert_type(loss_c, jnp.int32))

    def radix_body(r, accs):
        out = []
        for i in range(_IMGS):
            t_acc = accs[i]
            hi = jax.lax.shift_left(jnp.int32(1), 30 - 2 * r)
            lo = jax.lax.shift_left(jnp.int32(1), 29 - 2 * r)
            c1 = t_acc | lo
            c2 = t_acc | hi
            c3 = t_acc | hi | lo
            n1 = jnp.sum((us[i] >= c1).astype(jnp.int32))
            n2 = jnp.sum((us[i] >= c2).astype(jnp.int32))
            n3 = jnp.sum((us[i] >= c3).astype(jnp.int32))
            k = ks[i]
            t_acc = jnp.where(
                n3 >= k, c3,
                jnp.where(n2 >= k, c2, jnp.where(n1 >= k, c1, t_acc)))
            out.append(t_acc)
        return tuple(out)

    t_fins = jax.lax.fori_loop(0, 15, radix_body,
                               (jnp.int32(0),) * _IMGS)

    def last_body(accs):
        out = []
        for i in range(_IMGS):
            cand = accs[i] | 1
            cnt = jnp.sum((us[i] >= cand).astype(jnp.int32))
            out.append(jnp.where(cnt >= ks[i], cand, accs[i]))
        return tuple(out)

    t_fins = last_body(t_fins)

    for i in range(_IMGS):
        tau = jax.lax.bitcast_convert_type(t_fins[i], f32)
        loss_c = loss_cs[i]
        gt_mask = loss_c > tau
        sum_gt = jnp.sum(jnp.where(gt_mask, loss_c, 0.0))
        cnt_gt = jnp.sum(gt_mask.astype(jnp.int32))
        k = ks[i]
        topk = sum_gt + (k - cnt_gt).astype(f32) * tau
        topk = jnp.where(k > 0, topk, 0.0)
        o_l[i, :, :] = jnp.full((1, _L), loss_ls[i], f32)
        o_c[i, :, :] = jnp.full((1, _L), pos_ces[i], f32)
        o_k[i, :, :] = jnp.full((1, _L), topk, f32)
        o_n[i, :, :] = jnp.full((1, _L), num_poss[i].astype(f32), f32)


@jax.jit
def kernel(pred_loc, pred_score, priors, gt_data):
    f32 = jnp.float32
    pad = _PP - _P
    loc_t = jnp.pad(pred_loc.transpose(0, 2, 1), ((0, 0), (0, 0), (0, pad)))
    loc_t = loc_t.reshape(_B, 4, _R, _L)
    score_t = jnp.pad(pred_score.transpose(0, 2, 1),
                      ((0, 0), (0, 0), (0, pad)))
    score_t = score_t.reshape(_B, _C, _R, _L)
    pad_rows = jnp.tile(jnp.array([[0.5, 0.5, 1.0, 1.0]], f32), (pad, 1))
    pri_t = jnp.concatenate([priors, pad_rows], axis=0).T.reshape(4, _R, _L)

    grid = (_B // _IMGS,)
    out_shape = [jax.ShapeDtypeStruct((_B, 1, _L), f32)] * 4
    res = pl.pallas_call(
        _loss_kernel,
        grid=grid,
        in_specs=[
            pl.BlockSpec((_IMGS, _T, 5), lambda b: (b, 0, 0),
                         memory_space=pltpu.SMEM),
            pl.BlockSpec((_IMGS, 4, _R, _L), lambda b: (b, 0, 0, 0)),
            pl.BlockSpec((_IMGS, _C, _R, _L), lambda b: (b, 0, 0, 0)),
            pl.BlockSpec((4, _R, _L), lambda b: (0, 0, 0)),
        ],
        out_specs=[pl.BlockSpec((_IMGS, 1, _L), lambda b: (b, 0, 0))] * 4,
        out_shape=out_shape,
        compiler_params=pltpu.CompilerParams(
            dimension_semantics=("parallel",),
            vmem_limit_bytes=100 * 1024 * 1024),
    )(gt_data, loc_t, score_t, pri_t)
    l, c, k, n = (r[:, 0, 0] for r in res)
    n_tot = jnp.sum(n)
    return jnp.sum(l) / n_tot, (jnp.sum(c) + jnp.sum(k)) / n_tot


# speedup vs baseline: 1.3632x; 1.0370x over previous
import jax
import jax.numpy as jnp
from jax.experimental import pallas as pl
from jax.experimental.pallas import tpu as pltpu

_B, _P, _T, _C = 32, 24564, 50, 21
_L = 128
_R = (_P + _L - 1) // _L
_PP = _R * _L
_THRESH = 0.5
_NEG_POS = 3
_IMGS = 8


def _loss_kernel(gt_ref, loc_ref, score_ref, pri_ref, o_l, o_c, o_k, o_n):
    f32 = jnp.float32
    row = jax.lax.broadcasted_iota(jnp.int32, (_R, _L), 0)
    col = jax.lax.broadcasted_iota(jnp.int32, (_R, _L), 1)
    pidx = row * _L + col
    valid = pidx < _P

    cx = pri_ref[0]
    cy = pri_ref[1]
    pw = pri_ref[2]
    ph = pri_ref[3]
    px1 = cx - pw / 2.0
    py1 = cy - ph / 2.0
    px2 = cx + pw / 2.0
    py2 = cy + ph / 2.0
    area_p = (px2 - px1) * (py2 - py1)

    def match_one(i, t, bo, bi, fidx, fmask):
        tx1 = gt_ref[i, t, 0]
        ty1 = gt_ref[i, t, 1]
        tx2 = gt_ref[i, t, 2]
        ty2 = gt_ref[i, t, 3]
        iw = jnp.maximum(jnp.minimum(tx2, px2) - jnp.maximum(tx1, px1), 0.0)
        ih = jnp.maximum(jnp.minimum(ty2, py2) - jnp.maximum(ty1, py1), 0.0)
        inter = iw * ih
        area_t = (tx2 - tx1) * (ty2 - ty1)
        ov = inter / (area_t + area_p - inter)
        ov = jnp.where(valid, ov, -1.0)
        upd = ov > bo
        bo = jnp.where(upd, ov, bo)
        bi = jnp.where(upd, t, bi)
        m = jnp.max(ov)
        bp = jnp.min(jnp.where(ov == m, pidx, _PP))
        hit = pidx == bp
        fidx = jnp.where(hit, t, fidx)
        fmask = jnp.where(hit, 1, fmask)
        return bo, bi, fidx, fmask

    def match_body(t, carry):
        return tuple(match_one(i, t, *carry[i]) for i in range(_IMGS))

    one_init = (
        jnp.full((_R, _L), -jnp.inf, f32),
        jnp.zeros((_R, _L), jnp.int32),
        jnp.zeros((_R, _L), jnp.int32),
        jnp.zeros((_R, _L), jnp.int32),
    )
    mres = jax.lax.fori_loop(0, _T, match_body, (one_init,) * _IMGS)

    bis = []
    for i in range(_IMGS):
        bo, bi, fidx, fmask = mres[i]
        forced = fmask > 0
        bis.append((jnp.where(forced, 2.0, bo), jnp.where(forced, fidx, bi)))

    def gather_one(i, t, bi, lab, m1, m2, m3, m4):
        sel = bi == t
        lab = jnp.where(sel, gt_ref[i, t, 4], lab)
        m1 = jnp.where(sel, gt_ref[i, t, 0], m1)
        m2 = jnp.where(sel, gt_ref[i, t, 1], m2)
        m3 = jnp.where(sel, gt_ref[i, t, 2], m3)
        m4 = jnp.where(sel, gt_ref[i, t, 3], m4)
        return lab, m1, m2, m3, m4

    def gather_body(t, carry):
        return tuple(gather_one(i, t, bis[i][1], *carry[i])
                     for i in range(_IMGS))

    z = jnp.zeros((_R, _L), f32)
    gres = jax.lax.fori_loop(0, _T, gather_body, ((z,) * 5,) * _IMGS)

    def sl1(x):
        ax = jnp.abs(x)
        return jnp.where(ax < 1.0, 0.5 * x * x, ax - 0.5)

    confs, loss_ls = [], []
    for i in range(_IMGS):
        bo, _ = bis[i]
        lab, mx1, my1, mx2, my2 = gres[i]
        conf = jnp.where(bo < _THRESH, 0.0, lab + 1.0)
        conf = jnp.where(valid, conf, 0.0)
        confs.append(conf)
        g0 = ((mx1 + mx2) / 2.0 - cx) / (0.1 * pw)
        g1 = ((my1 + my2) / 2.0 - cy) / (0.1 * ph)
        g2 = jnp.log((mx2 - mx1) / pw) / 0.2
        g3 = jnp.log((my2 - my1) / ph) / 0.2
        lsum = (sl1(loc_ref[i, 0] - g0) + sl1(loc_ref[i, 1] - g1)
                + sl1(loc_ref[i, 2] - g2) + sl1(loc_ref[i, 3] - g3))
        loss_ls.append(jnp.sum(jnp.where(conf > 0.0, lsum, 0.0)))

    ms = [score_ref[i, 0] for i in range(_IMGS)]
    for c in range(1, _C):
        ms = [jnp.maximum(ms[i], score_ref[i, c]) for i in range(_IMGS)]
    ss = [jnp.zeros((_R, _L), f32) for _ in range(_IMGS)]
    pks = [jnp.zeros((_R, _L), f32) for _ in range(_IMGS)]
    for c in range(_C):
        for i in range(_IMGS):
            x = score_ref[i, c]
            ss[i] = ss[i] + jnp.exp(x - ms[i])
            pks[i] = jnp.where(confs[i] == c, x, pks[i])

    ces, loss_cs, us, pos_ces, num_poss, ks = [], [], [], [], [], []
    for i in range(_IMGS):
        ce = ms[i] + jnp.log(ss[i]) - pks[i]
        pos = confs[i] > 0.0
        pos_ces.append(jnp.sum(jnp.where(pos, ce, 0.0)))
        num_pos = jnp.sum(pos.astype(jnp.int32))
        num_poss.append(num_pos)
        ks.append(jnp.minimum(_NEG_POS * num_pos, _P - 1))
        loss_c = jnp.where(pos, 0.0, ce)
        loss_c = jnp.where(valid, loss_c, 0.0)
        loss_cs.append(loss_c)
        us.append(jax.lax.bitcast_convert_type(loss_c, jnp.int32))

    def radix_body(r, accs):
        out = []
        for i in range(_IMGS):
            t_acc = accs[i]
            hi = jax.lax.shift_left(jnp.int32(1), 30 - 2 * r)
            lo = jax.lax.shift_left(jnp.int32(1), 29 - 2 * r)
            c1 = t_acc | lo
            c2 = t_acc | hi
            c3 = t_acc | hi | lo
            n1 = jnp.sum((us[i] >= c1).astype(jnp.int32))
            n2 = jnp.sum((us[i] >= c2).astype(jnp.int32))
            n3 = jnp.sum((us[i] >= c3).astype(jnp.int32))
            k = ks[i]
            t_acc = jnp.where(
                n3 >= k, c3,
                jnp.where(n2 >= k, c2, jnp.where(n1 >= k, c1, t_acc)))
            out.append(t_acc)
        return tuple(out)

    t_fins = jax.lax.fori_loop(0, 15, radix_body,
                               (jnp.int32(0),) * _IMGS)

    def last_body(accs):
        out = []
        for i in range(_IMGS):
            cand = accs[i] | 1
            cnt = jnp.sum((us[i] >= cand).astype(jnp.int32))
            out.append(jnp.where(cnt >= ks[i], cand, accs[i]))
        return tuple(out)

    t_fins = last_body(t_fins)

    for i in range(_IMGS):
        tau = jax.lax.bitcast_convert_type(t_fins[i], f32)
        loss_c = loss_cs[i]
        gt_mask = loss_c > tau
        sum_gt = jnp.sum(jnp.where(gt_mask, loss_c, 0.0))
        cnt_gt = jnp.sum(gt_mask.astype(jnp.int32))
        k = ks[i]
        topk = sum_gt + (k - cnt_gt).astype(f32) * tau
        topk = jnp.where(k > 0, topk, 0.0)
        o_l[i, :, :] = jnp.full((1, _L), loss_ls[i], f32)
        o_c[i, :, :] = jnp.full((1, _L), pos_ces[i], f32)
        o_k[i, :, :] = jnp.full((1, _L), topk, f32)
        o_n[i, :, :] = jnp.full((1, _L), num_poss[i].astype(f32), f32)


@jax.jit
def kernel(pred_loc, pred_score, priors, gt_data):
    f32 = jnp.float32
    pad = _PP - _P
    loc_t = jnp.pad(pred_loc.transpose(0, 2, 1), ((0, 0), (0, 0), (0, pad)))
    loc_t = loc_t.reshape(_B, 4, _R, _L)
    score_t = jnp.pad(pred_score.transpose(0, 2, 1),
                      ((0, 0), (0, 0), (0, pad)))
    score_t = score_t.reshape(_B, _C, _R, _L)
    pad_rows = jnp.tile(jnp.array([[0.5, 0.5, 1.0, 1.0]], f32), (pad, 1))
    pri_t = jnp.concatenate([priors, pad_rows], axis=0).T.reshape(4, _R, _L)

    grid = (_B // _IMGS,)
    out_shape = [jax.ShapeDtypeStruct((_B, 1, _L), f32)] * 4
    res = pl.pallas_call(
        _loss_kernel,
        grid=grid,
        in_specs=[
            pl.BlockSpec((_IMGS, _T, 5), lambda b: (b, 0, 0),
                         memory_space=pltpu.SMEM),
            pl.BlockSpec((_IMGS, 4, _R, _L), lambda b: (b, 0, 0, 0)),
            pl.BlockSpec((_IMGS, _C, _R, _L), lambda b: (b, 0, 0, 0)),
            pl.BlockSpec((4, _R, _L), lambda b: (0, 0, 0)),
        ],
        out_specs=[pl.BlockSpec((_IMGS, 1, _L), lambda b: (b, 0, 0))] * 4,
        out_shape=out_shape,
        compiler_params=pltpu.CompilerParams(
            dimension_semantics=("parallel",),
            vmem_limit_bytes=100 * 1024 * 1024),
    )(gt_data, loc_t, score_t, pri_t)
    l, c, k, n = (r[:, 0, 0] for r in res)
    n_tot = jnp.sum(n)
    return jnp.sum(l) / n_tot, (jnp.sum(c) + jnp.sum(k)) / n_tot


# speedup vs baseline: 1.4273x; 1.0470x over previous
import jax
import jax.numpy as jnp
from jax.experimental import pallas as pl
from jax.experimental.pallas import tpu as pltpu

_B, _P, _T, _C = 32, 24564, 50, 21
_L = 128
_R = (_P + _L - 1) // _L
_PP = _R * _L
_THRESH = 0.5
_NEG_POS = 3
_IMGS = 8


def _loss_kernel(gt_ref, loc_ref, score_ref, pri_ref, o_l, o_c, o_k, o_n):
    f32 = jnp.float32
    row = jax.lax.broadcasted_iota(jnp.int32, (_R, _L), 0)
    col = jax.lax.broadcasted_iota(jnp.int32, (_R, _L), 1)
    pidx = row * _L + col
    valid = pidx < _P

    cx = pri_ref[0]
    cy = pri_ref[1]
    pw = pri_ref[2]
    ph = pri_ref[3]
    px1 = cx - pw / 2.0
    py1 = cy - ph / 2.0
    px2 = cx + pw / 2.0
    py2 = cy + ph / 2.0
    area_p = (px2 - px1) * (py2 - py1)

    def match_one(i, t, bo, bi, fidx):
        tx1 = gt_ref[i, t, 0]
        ty1 = gt_ref[i, t, 1]
        tx2 = gt_ref[i, t, 2]
        ty2 = gt_ref[i, t, 3]
        iw = jnp.maximum(jnp.minimum(tx2, px2) - jnp.maximum(tx1, px1), 0.0)
        ih = jnp.maximum(jnp.minimum(ty2, py2) - jnp.maximum(ty1, py1), 0.0)
        inter = iw * ih
        area_t = (tx2 - tx1) * (ty2 - ty1)
        ov = inter / (area_t + area_p - inter)
        upd = ov > bo
        bo = jnp.where(upd, ov, bo)
        bi = jnp.where(upd, t, bi)
        m = jnp.max(ov)
        bp = jnp.min(jnp.where(ov == m, pidx, _PP))
        fidx = jnp.where(pidx == bp, t, fidx)
        return bo, bi, fidx

    def match_body(t, carry):
        return tuple(match_one(i, t, *carry[i]) for i in range(_IMGS))

    one_init = (
        jnp.full((_R, _L), -jnp.inf, f32),
        jnp.zeros((_R, _L), jnp.int32),
        jnp.full((_R, _L), -1, jnp.int32),
    )
    mres = jax.lax.fori_loop(0, _T, match_body, (one_init,) * _IMGS)

    bis = []
    for i in range(_IMGS):
        bo, bi, fidx = mres[i]
        forced = fidx >= 0
        bis.append((jnp.where(forced, 2.0, bo), jnp.where(forced, fidx, bi)))

    def gather_one(i, t, bi, lab, m1, m2, m3, m4):
        sel = bi == t
        lab = jnp.where(sel, gt_ref[i, t, 4], lab)
        m1 = jnp.where(sel, gt_ref[i, t, 0], m1)
        m2 = jnp.where(sel, gt_ref[i, t, 1], m2)
        m3 = jnp.where(sel, gt_ref[i, t, 2], m3)
        m4 = jnp.where(sel, gt_ref[i, t, 3], m4)
        return lab, m1, m2, m3, m4

    def gather_body(t, carry):
        return tuple(gather_one(i, t, bis[i][1], *carry[i])
                     for i in range(_IMGS))

    z = jnp.zeros((_R, _L), f32)
    gres = jax.lax.fori_loop(0, _T, gather_body, ((z,) * 5,) * _IMGS)

    def sl1(x):
        ax = jnp.abs(x)
        return jnp.where(ax < 1.0, 0.5 * x * x, ax - 0.5)

    confs, loss_ls = [], []
    for i in range(_IMGS):
        bo, _ = bis[i]
        lab, mx1, my1, mx2, my2 = gres[i]
        conf = jnp.where(bo < _THRESH, 0.0, lab + 1.0)
        confs.append(conf)
        g0 = ((mx1 + mx2) / 2.0 - cx) / (0.1 * pw)
        g1 = ((my1 + my2) / 2.0 - cy) / (0.1 * ph)
        g2 = jnp.log((mx2 - mx1) / pw) / 0.2
        g3 = jnp.log((my2 - my1) / ph) / 0.2
        lsum = (sl1(loc_ref[i, 0] - g0) + sl1(loc_ref[i, 1] - g1)
                + sl1(loc_ref[i, 2] - g2) + sl1(loc_ref[i, 3] - g3))
        loss_ls.append(jnp.sum(jnp.where(conf > 0.0, lsum, 0.0)))

    ms = [score_ref[i, 0] for i in range(_IMGS)]
    for c in range(1, _C):
        ms = [jnp.maximum(ms[i], score_ref[i, c]) for i in range(_IMGS)]
    ss = [jnp.zeros((_R, _L), f32) for _ in range(_IMGS)]
    pks = [jnp.zeros((_R, _L), f32) for _ in range(_IMGS)]
    for c in range(_C):
        for i in range(_IMGS):
            x = score_ref[i, c]
            ss[i] = ss[i] + jnp.exp(x - ms[i])
            pks[i] = jnp.where(confs[i] == c, x, pks[i])

    ces, loss_cs, us, pos_ces, num_poss, ks = [], [], [], [], [], []
    for i in range(_IMGS):
        ce = ms[i] + jnp.log(ss[i]) - pks[i]
        pos = confs[i] > 0.0
        pos_ces.append(jnp.sum(jnp.where(pos, ce, 0.0)))
        num_pos = jnp.sum(pos.astype(jnp.int32))
        num_poss.append(num_pos)
        ks.append(jnp.minimum(_NEG_POS * num_pos, _P - 1))
        loss_c = jnp.where(pos, 0.0, ce)
        loss_c = jnp.where(valid, loss_c, 0.0)
        loss_cs.append(loss_c)
        us.append(jax.lax.bitcast_convert_type(loss_c, jnp.int32))

    def radix_body(r, accs):
        out = []
        for i in range(_IMGS):
            t_acc = accs[i]
            hi = jax.lax.shift_left(jnp.int32(1), 30 - 2 * r)
            lo = jax.lax.shift_left(jnp.int32(1), 29 - 2 * r)
            c1 = t_acc | lo
            c2 = t_acc | hi
            c3 = t_acc | hi | lo
            n1 = jnp.sum((us[i] >= c1).astype(jnp.int32))
            n2 = jnp.sum((us[i] >= c2).astype(jnp.int32))
            n3 = jnp.sum((us[i] >= c3).astype(jnp.int32))
            k = ks[i]
            t_acc = jnp.where(
                n3 >= k, c3,
                jnp.where(n2 >= k, c2, jnp.where(n1 >= k, c1, t_acc)))
            out.append(t_acc)
        return tuple(out)

    t_fins = jax.lax.fori_loop(0, 15, radix_body,
                               (jnp.int32(0),) * _IMGS)

    def last_body(accs):
        out = []
        for i in range(_IMGS):
            cand = accs[i] | 1
            cnt = jnp.sum((us[i] >= cand).astype(jnp.int32))
            out.append(jnp.where(cnt >= ks[i], cand, accs[i]))
        return tuple(out)

    t_fins = last_body(t_fins)

    for i in range(_IMGS):
        tau = jax.lax.bitcast_convert_type(t_fins[i], f32)
        loss_c = loss_cs[i]
        gt_mask = loss_c > tau
        sum_gt = jnp.sum(jnp.where(gt_mask, loss_c, 0.0))
        cnt_gt = jnp.sum(gt_mask.astype(jnp.int32))
        k = ks[i]
        topk = sum_gt + (k - cnt_gt).astype(f32) * tau
        topk = jnp.where(k > 0, topk, 0.0)
        o_l[i, :, :] = jnp.full((1, _L), loss_ls[i], f32)
        o_c[i, :, :] = jnp.full((1, _L), pos_ces[i], f32)
        o_k[i, :, :] = jnp.full((1, _L), topk, f32)
        o_n[i, :, :] = jnp.full((1, _L), num_poss[i].astype(f32), f32)


@jax.jit
def kernel(pred_loc, pred_score, priors, gt_data):
    f32 = jnp.float32
    pad = _PP - _P
    loc_t = jnp.pad(pred_loc.transpose(0, 2, 1), ((0, 0), (0, 0), (0, pad)))
    loc_t = loc_t.reshape(_B, 4, _R, _L)
    score_t = jnp.pad(pred_score.transpose(0, 2, 1),
                      ((0, 0), (0, 0), (0, pad)))
    score_t = score_t.reshape(_B, _C, _R, _L)
    pad_rows = jnp.tile(jnp.array([[-100.0, -100.0, 0.0, 0.0]], f32),
                        (pad, 1))
    pri_t = jnp.concatenate([priors, pad_rows], axis=0).T.reshape(4, _R, _L)

    grid = (_B // _IMGS,)
    out_shape = [jax.ShapeDtypeStruct((_B, 1, _L), f32)] * 4
    res = pl.pallas_call(
        _loss_kernel,
        grid=grid,
        in_specs=[
            pl.BlockSpec((_IMGS, _T, 5), lambda b: (b, 0, 0),
                         memory_space=pltpu.SMEM),
            pl.BlockSpec((_IMGS, 4, _R, _L), lambda b: (b, 0, 0, 0)),
            pl.BlockSpec((_IMGS, _C, _R, _L), lambda b: (b, 0, 0, 0)),
            pl.BlockSpec((4, _R, _L), lambda b: (0, 0, 0)),
        ],
        out_specs=[pl.BlockSpec((_IMGS, 1, _L), lambda b: (b, 0, 0))] * 4,
        out_shape=out_shape,
        compiler_params=pltpu.CompilerParams(
            dimension_semantics=("parallel",),
            vmem_limit_bytes=100 * 1024 * 1024),
    )(gt_data, loc_t, score_t, pri_t)
    l, c, k, n = (r[:, 0, 0] for r in res)
    n_tot = jnp.sum(n)
    return jnp.sum(l) / n_tot, (jnp.sum(c) + jnp.sum(k)) / n_tot


# speedup vs baseline: 1.6115x; 1.1291x over previous
import jax
import jax.numpy as jnp
from jax.experimental import pallas as pl
from jax.experimental.pallas import tpu as pltpu

_B, _P, _T, _C = 32, 24564, 50, 21
_L = 128
_R = (_P + _L - 1) // _L
_PP = _R * _L
_THRESH = 0.5
_NEG_POS = 3
_IMGS = 8


def _loss_kernel(gt_ref, loc_ref, score_ref, pri_ref, o_l, o_c, o_k, o_n):
    f32 = jnp.float32
    row = jax.lax.broadcasted_iota(jnp.int32, (_R, _L), 0)
    col = jax.lax.broadcasted_iota(jnp.int32, (_R, _L), 1)
    pidx = row * _L + col
    valid = pidx < _P

    cx = pri_ref[0]
    cy = pri_ref[1]
    pw = pri_ref[2]
    ph = pri_ref[3]
    px1 = cx - pw / 2.0
    py1 = cy - ph / 2.0
    px2 = cx + pw / 2.0
    py2 = cy + ph / 2.0
    area_p = (px2 - px1) * (py2 - py1)

    def match_one(i, t, bo, bi, fidx):
        tx1 = gt_ref[i, t, 0]
        ty1 = gt_ref[i, t, 1]
        tx2 = gt_ref[i, t, 2]
        ty2 = gt_ref[i, t, 3]
        iw = jnp.maximum(jnp.minimum(tx2, px2) - jnp.maximum(tx1, px1), 0.0)
        ih = jnp.maximum(jnp.minimum(ty2, py2) - jnp.maximum(ty1, py1), 0.0)
        inter = iw * ih
        area_t = (tx2 - tx1) * (ty2 - ty1)
        ov = inter / (area_t + area_p - inter)
        upd = ov > bo
        bo = jnp.where(upd, ov, bo)
        bi = jnp.where(upd, t, bi)
        m = jnp.max(ov)
        bp = jnp.min(jnp.where(ov == m, pidx, _PP))
        fidx = jnp.where(pidx == bp, t, fidx)
        return bo, bi, fidx

    def match_body(it, carry):
        t = it * 2
        carry = tuple(match_one(i, t, *carry[i]) for i in range(_IMGS))
        return tuple(match_one(i, t + 1, *carry[i]) for i in range(_IMGS))

    one_init = (
        jnp.full((_R, _L), -jnp.inf, f32),
        jnp.zeros((_R, _L), jnp.int32),
        jnp.full((_R, _L), -1, jnp.int32),
    )
    mres = jax.lax.fori_loop(0, _T // 2, match_body, (one_init,) * _IMGS)

    bis = []
    for i in range(_IMGS):
        bo, bi, fidx = mres[i]
        forced = fidx >= 0
        bis.append((jnp.where(forced, 2.0, bo), jnp.where(forced, fidx, bi)))

    def gather_one(i, t, bi, lab, m1, m2, m3, m4):
        sel = bi == t
        lab = jnp.where(sel, gt_ref[i, t, 4], lab)
        m1 = jnp.where(sel, gt_ref[i, t, 0], m1)
        m2 = jnp.where(sel, gt_ref[i, t, 1], m2)
        m3 = jnp.where(sel, gt_ref[i, t, 2], m3)
        m4 = jnp.where(sel, gt_ref[i, t, 3], m4)
        return lab, m1, m2, m3, m4

    def gather_body(it, carry):
        t = it * 2
        carry = tuple(gather_one(i, t, bis[i][1], *carry[i])
                      for i in range(_IMGS))
        return tuple(gather_one(i, t + 1, bis[i][1], *carry[i])
                     for i in range(_IMGS))

    z = jnp.zeros((_R, _L), f32)
    gres = jax.lax.fori_loop(0, _T // 2, gather_body, ((z,) * 5,) * _IMGS)

    def sl1(x):
        ax = jnp.abs(x)
        return jnp.where(ax < 1.0, 0.5 * x * x, ax - 0.5)

    confs, loss_ls = [], []
    for i in range(_IMGS):
        bo, _ = bis[i]
        lab, mx1, my1, mx2, my2 = gres[i]
        conf = jnp.where(bo < _THRESH, 0.0, lab + 1.0)
        confs.append(conf)
        g0 = ((mx1 + mx2) / 2.0 - cx) / (0.1 * pw)
        g1 = ((my1 + my2) / 2.0 - cy) / (0.1 * ph)
        g2 = jnp.log((mx2 - mx1) / pw) / 0.2
        g3 = jnp.log((my2 - my1) / ph) / 0.2
        lsum = (sl1(loc_ref[i, 0] - g0) + sl1(loc_ref[i, 1] - g1)
                + sl1(loc_ref[i, 2] - g2) + sl1(loc_ref[i, 3] - g3))
        loss_ls.append(jnp.sum(jnp.where(conf > 0.0, lsum, 0.0)))

    ms = [score_ref[i, 0] for i in range(_IMGS)]
    for c in range(1, _C):
        ms = [jnp.maximum(ms[i], score_ref[i, c]) for i in range(_IMGS)]
    ss = [jnp.zeros((_R, _L), f32) for _ in range(_IMGS)]
    pks = [jnp.zeros((_R, _L), f32) for _ in range(_IMGS)]
    for c in range(_C):
        for i in range(_IMGS):
            x = score_ref[i, c]
            ss[i] = ss[i] + jnp.exp(x - ms[i])
            pks[i] = jnp.where(confs[i] == c, x, pks[i])

    ces, loss_cs, us, pos_ces, num_poss, ks = [], [], [], [], [], []
    for i in range(_IMGS):
        ce = ms[i] + jnp.log(ss[i]) - pks[i]
        pos = confs[i] > 0.0
        pos_ces.append(jnp.sum(jnp.where(pos, ce, 0.0)))
        num_pos = jnp.sum(pos.astype(jnp.int32))
        num_poss.append(num_pos)
        ks.append(jnp.minimum(_NEG_POS * num_pos, _P - 1))
        loss_c = jnp.where(pos, 0.0, ce)
        loss_c = jnp.where(valid, loss_c, 0.0)
        loss_cs.append(loss_c)
        us.append(jax.lax.bitcast_convert_type(loss_c, jnp.int32))

    def radix_body(r, accs):
        out = []
        for i in range(_IMGS):
            t_acc = accs[i]
            hi = jax.lax.shift_left(jnp.int32(1), 30 - 2 * r)
            lo = jax.lax.shift_left(jnp.int32(1), 29 - 2 * r)
            c1 = t_acc | lo
            c2 = t_acc | hi
            c3 = t_acc | hi | lo
            n1 = jnp.sum((us[i] >= c1).astype(jnp.int32))
            n2 = jnp.sum((us[i] >= c2).astype(jnp.int32))
            n3 = jnp.sum((us[i] >= c3).astype(jnp.int32))
            k = ks[i]
            t_acc = jnp.where(
                n3 >= k, c3,
                jnp.where(n2 >= k, c2, jnp.where(n1 >= k, c1, t_acc)))
            out.append(t_acc)
        return tuple(out)

    t_fins = jax.lax.fori_loop(0, 15, radix_body,
                               (jnp.int32(0),) * _IMGS)

    def last_body(accs):
        out = []
        for i in range(_IMGS):
            cand = accs[i] | 1
            cnt = jnp.sum((us[i] >= cand).astype(jnp.int32))
            out.append(jnp.where(cnt >= ks[i], cand, accs[i]))
        return tuple(out)

    t_fins = last_body(t_fins)

    for i in range(_IMGS):
        tau = jax.lax.bitcast_convert_type(t_fins[i], f32)
        loss_c = loss_cs[i]
        gt_mask = loss_c > tau
        sum_gt = jnp.sum(jnp.where(gt_mask, loss_c, 0.0))
        cnt_gt = jnp.sum(gt_mask.astype(jnp.int32))
        k = ks[i]
        topk = sum_gt + (k - cnt_gt).astype(f32) * tau
        topk = jnp.where(k > 0, topk, 0.0)
        o_l[i, :, :] = jnp.full((1, _L), loss_ls[i], f32)
        o_c[i, :, :] = jnp.full((1, _L), pos_ces[i], f32)
        o_k[i, :, :] = jnp.full((1, _L), topk, f32)
        o_n[i, :, :] = jnp.full((1, _L), num_poss[i].astype(f32), f32)


@jax.jit
def kernel(pred_loc, pred_score, priors, gt_data):
    f32 = jnp.float32
    pad = _PP - _P
    loc_t = jnp.pad(pred_loc.transpose(0, 2, 1), ((0, 0), (0, 0), (0, pad)))
    loc_t = loc_t.reshape(_B, 4, _R, _L)
    score_t = jnp.pad(pred_score.transpose(0, 2, 1),
                      ((0, 0), (0, 0), (0, pad)))
    score_t = score_t.reshape(_B, _C, _R, _L)
    pad_rows = jnp.tile(jnp.array([[-100.0, -100.0, 0.0, 0.0]], f32),
                        (pad, 1))
    pri_t = jnp.concatenate([priors, pad_rows], axis=0).T.reshape(4, _R, _L)

    grid = (_B // _IMGS,)
    out_shape = [jax.ShapeDtypeStruct((_B, 1, _L), f32)] * 4
    res = pl.pallas_call(
        _loss_kernel,
        grid=grid,
        in_specs=[
            pl.BlockSpec((_IMGS, _T, 5), lambda b: (b, 0, 0),
                         memory_space=pltpu.SMEM),
            pl.BlockSpec((_IMGS, 4, _R, _L), lambda b: (b, 0, 0, 0)),
            pl.BlockSpec((_IMGS, _C, _R, _L), lambda b: (b, 0, 0, 0)),
            pl.BlockSpec((4, _R, _L), lambda b: (0, 0, 0)),
        ],
        out_specs=[pl.BlockSpec((_IMGS, 1, _L), lambda b: (b, 0, 0))] * 4,
        out_shape=out_shape,
        compiler_params=pltpu.CompilerParams(
            dimension_semantics=("parallel",),
            vmem_limit_bytes=100 * 1024 * 1024),
    )(gt_data, loc_t, score_t, pri_t)
    l, c, k, n = (r[:, 0, 0] for r in res)
    n_tot = jnp.sum(n)
    return jnp.sum(l) / n_tot, (jnp.sum(c) + jnp.sum(k)) / n_tot


# speedup vs baseline: 1.7137x; 1.0634x over previous
import jax
import jax.numpy as jnp
from jax.experimental import pallas as pl
from jax.experimental.pallas import tpu as pltpu

_B, _P, _T, _C = 32, 24564, 50, 21
_L = 128
_R = (_P + _L - 1) // _L
_PP = _R * _L
_THRESH = 0.5
_NEG_POS = 3
_IMGS = 8


def _loss_kernel(gt_ref, loc_ref, score_ref, pri_ref, o_l, o_c, o_k, o_n):
    f32 = jnp.float32
    row = jax.lax.broadcasted_iota(jnp.int32, (_R, _L), 0)
    col = jax.lax.broadcasted_iota(jnp.int32, (_R, _L), 1)
    pidx = row * _L + col
    valid = pidx < _P

    cx = pri_ref[0]
    cy = pri_ref[1]
    pw = pri_ref[2]
    ph = pri_ref[3]
    px1 = cx - pw / 2.0
    py1 = cy - ph / 2.0
    px2 = cx + pw / 2.0
    py2 = cy + ph / 2.0
    area_p = (px2 - px1) * (py2 - py1)

    def match_one(i, t, bo, bi, fidx):
        tx1 = gt_ref[i, t, 0]
        ty1 = gt_ref[i, t, 1]
        tx2 = gt_ref[i, t, 2]
        ty2 = gt_ref[i, t, 3]
        iw = jnp.maximum(jnp.minimum(tx2, px2) - jnp.maximum(tx1, px1), 0.0)
        ih = jnp.maximum(jnp.minimum(ty2, py2) - jnp.maximum(ty1, py1), 0.0)
        inter = iw * ih
        area_t = (tx2 - tx1) * (ty2 - ty1)
        ov = inter / (area_t + area_p - inter)
        upd = ov > bo
        bo = jnp.where(upd, ov, bo)
        bi = jnp.where(upd, t, bi)
        m = jnp.max(ov)
        bp = jnp.min(jnp.where(ov == m, pidx, _PP))
        fidx = jnp.where(pidx == bp, t, fidx)
        return bo, bi, fidx

    _UNROLL = 5

    def match_body(it, carry):
        for u in range(_UNROLL):
            t = it * _UNROLL + u
            carry = tuple(match_one(i, t, *carry[i]) for i in range(_IMGS))
        return carry

    one_init = (
        jnp.full((_R, _L), -jnp.inf, f32),
        jnp.zeros((_R, _L), jnp.int32),
        jnp.full((_R, _L), -1, jnp.int32),
    )
    mres = jax.lax.fori_loop(0, _T // _UNROLL, match_body,
                             (one_init,) * _IMGS)

    bis = []
    for i in range(_IMGS):
        bo, bi, fidx = mres[i]
        forced = fidx >= 0
        bis.append((jnp.where(forced, 2.0, bo), jnp.where(forced, fidx, bi)))

    def gather_one(i, t, bi, lab, m1, m2, m3, m4):
        sel = bi == t
        lab = jnp.where(sel, gt_ref[i, t, 4], lab)
        m1 = jnp.where(sel, gt_ref[i, t, 0], m1)
        m2 = jnp.where(sel, gt_ref[i, t, 1], m2)
        m3 = jnp.where(sel, gt_ref[i, t, 2], m3)
        m4 = jnp.where(sel, gt_ref[i, t, 3], m4)
        return lab, m1, m2, m3, m4

    def gather_body(it, carry):
        for u in range(_UNROLL):
            t = it * _UNROLL + u
            carry = tuple(gather_one(i, t, bis[i][1], *carry[i])
                          for i in range(_IMGS))
        return carry

    z = jnp.zeros((_R, _L), f32)
    gres = jax.lax.fori_loop(0, _T // _UNROLL, gather_body,
                             ((z,) * 5,) * _IMGS)

    def sl1(x):
        ax = jnp.abs(x)
        return jnp.where(ax < 1.0, 0.5 * x * x, ax - 0.5)

    confs, loss_ls = [], []
    for i in range(_IMGS):
        bo, _ = bis[i]
        lab, mx1, my1, mx2, my2 = gres[i]
        conf = jnp.where(bo < _THRESH, 0.0, lab + 1.0)
        confs.append(conf)
        g0 = ((mx1 + mx2) / 2.0 - cx) / (0.1 * pw)
        g1 = ((my1 + my2) / 2.0 - cy) / (0.1 * ph)
        g2 = jnp.log((mx2 - mx1) / pw) / 0.2
        g3 = jnp.log((my2 - my1) / ph) / 0.2
        lsum = (sl1(loc_ref[i, 0] - g0) + sl1(loc_ref[i, 1] - g1)
                + sl1(loc_ref[i, 2] - g2) + sl1(loc_ref[i, 3] - g3))
        loss_ls.append(jnp.sum(jnp.where(conf > 0.0, lsum, 0.0)))

    ms = [score_ref[i, 0] for i in range(_IMGS)]
    for c in range(1, _C):
        ms = [jnp.maximum(ms[i], score_ref[i, c]) for i in range(_IMGS)]
    ss = [jnp.zeros((_R, _L), f32) for _ in range(_IMGS)]
    pks = [jnp.zeros((_R, _L), f32) for _ in range(_IMGS)]
    for c in range(_C):
        for i in range(_IMGS):
            x = score_ref[i, c]
            ss[i] = ss[i] + jnp.exp(x - ms[i])
            pks[i] = jnp.where(confs[i] == c, x, pks[i])

    ces, loss_cs, us, pos_ces, num_poss, ks = [], [], [], [], [], []
    for i in range(_IMGS):
        ce = ms[i] + jnp.log(ss[i]) - pks[i]
        pos = confs[i] > 0.0
        pos_ces.append(jnp.sum(jnp.where(pos, ce, 0.0)))
        num_pos = jnp.sum(pos.astype(jnp.int32))
        num_poss.append(num_pos)
        ks.append(jnp.minimum(_NEG_POS * num_pos, _P - 1))
        loss_c = jnp.where(pos, 0.0, ce)
        loss_c = jnp.where(valid, loss_c, 0.0)
        loss_cs.append(loss_c)
        us.append(jax.lax.bitcast_convert_type(loss_c, jnp.int32))

    def radix_body(r, accs):
        out = []
        for i in range(_IMGS):
            t_acc = accs[i]
            hi = jax.lax.shift_left(jnp.int32(1), 30 - 2 * r)
            lo = jax.lax.shift_left(jnp.int32(1), 29 - 2 * r)
            c1 = t_acc | lo
            c2 = t_acc | hi
            c3 = t_acc | hi | lo
            n1 = jnp.sum((us[i] >= c1).astype(jnp.int32))
            n2 = jnp.sum((us[i] >= c2).astype(jnp.int32))
            n3 = jnp.sum((us[i] >= c3).astype(jnp.int32))
            k = ks[i]
            t_acc = jnp.where(
                n3 >= k, c3,
                jnp.where(n2 >= k, c2, jnp.where(n1 >= k, c1, t_acc)))
            out.append(t_acc)
        return tuple(out)

    t_fins = jax.lax.fori_loop(0, 15, radix_body,
                               (jnp.int32(0),) * _IMGS)

    def last_body(accs):
        out = []
        for i in range(_IMGS):
            cand = accs[i] | 1
            cnt = jnp.sum((us[i] >= cand).astype(jnp.int32))
            out.append(jnp.where(cnt >= ks[i], cand, accs[i]))
        return tuple(out)

    t_fins = last_body(t_fins)

    for i in range(_IMGS):
        tau = jax.lax.bitcast_convert_type(t_fins[i], f32)
        loss_c = loss_cs[i]
        gt_mask = loss_c > tau
        sum_gt = jnp.sum(jnp.where(gt_mask, loss_c, 0.0))
        cnt_gt = jnp.sum(gt_mask.astype(jnp.int32))
        k = ks[i]
        topk = sum_gt + (k - cnt_gt).astype(f32) * tau
        topk = jnp.where(k > 0, topk, 0.0)
        o_l[i, :, :] = jnp.full((1, _L), loss_ls[i], f32)
        o_c[i, :, :] = jnp.full((1, _L), pos_ces[i], f32)
        o_k[i, :, :] = jnp.full((1, _L), topk, f32)
        o_n[i, :, :] = jnp.full((1, _L), num_poss[i].astype(f32), f32)


@jax.jit
def kernel(pred_loc, pred_score, priors, gt_data):
    f32 = jnp.float32
    pad = _PP - _P
    loc_t = jnp.pad(pred_loc.transpose(0, 2, 1), ((0, 0), (0, 0), (0, pad)))
    loc_t = loc_t.reshape(_B, 4, _R, _L)
    score_t = jnp.pad(pred_score.transpose(0, 2, 1),
                      ((0, 0), (0, 0), (0, pad)))
    score_t = score_t.reshape(_B, _C, _R, _L)
    pad_rows = jnp.tile(jnp.array([[-100.0, -100.0, 0.0, 0.0]], f32),
                        (pad, 1))
    pri_t = jnp.concatenate([priors, pad_rows], axis=0).T.reshape(4, _R, _L)

    grid = (_B // _IMGS,)
    out_shape = [jax.ShapeDtypeStruct((_B, 1, _L), f32)] * 4
    res = pl.pallas_call(
        _loss_kernel,
        grid=grid,
        in_specs=[
            pl.BlockSpec((_IMGS, _T, 5), lambda b: (b, 0, 0),
                         memory_space=pltpu.SMEM),
            pl.BlockSpec((_IMGS, 4, _R, _L), lambda b: (b, 0, 0, 0)),
            pl.BlockSpec((_IMGS, _C, _R, _L), lambda b: (b, 0, 0, 0)),
            pl.BlockSpec((4, _R, _L), lambda b: (0, 0, 0)),
        ],
        out_specs=[pl.BlockSpec((_IMGS, 1, _L), lambda b: (b, 0, 0))] * 4,
        out_shape=out_shape,
        compiler_params=pltpu.CompilerParams(
            dimension_semantics=("parallel",),
            vmem_limit_bytes=100 * 1024 * 1024),
    )(gt_data, loc_t, score_t, pri_t)
    l, c, k, n = (r[:, 0, 0] for r in res)
    n_tot = jnp.sum(n)
    return jnp.sum(l) / n_tot, (jnp.sum(c) + jnp.sum(k)) / n_tot


# speedup vs baseline: 1.7700x; 1.0329x over previous
import jax
import jax.numpy as jnp
from jax.experimental import pallas as pl
from jax.experimental.pallas import tpu as pltpu

_B, _P, _T, _C = 32, 24564, 50, 21
_L = 128
_R = (_P + _L - 1) // _L
_PP = _R * _L
_THRESH = 0.5
_NEG_POS = 3
_IMGS = 8


def _loss_kernel(gt_ref, loc_ref, score_ref, pri_ref, o_l, o_c, o_k, o_n):
    f32 = jnp.float32
    row = jax.lax.broadcasted_iota(jnp.int32, (_R, _L), 0)
    col = jax.lax.broadcasted_iota(jnp.int32, (_R, _L), 1)
    pidx = row * _L + col
    valid = pidx < _P

    cx = pri_ref[0]
    cy = pri_ref[1]
    pw = pri_ref[2]
    ph = pri_ref[3]
    px1 = cx - pw / 2.0
    py1 = cy - ph / 2.0
    px2 = cx + pw / 2.0
    py2 = cy + ph / 2.0
    area_p = (px2 - px1) * (py2 - py1)

    def match_one(i, t, bo, bi, fidx):
        tx1 = gt_ref[i, t, 0]
        ty1 = gt_ref[i, t, 1]
        tx2 = gt_ref[i, t, 2]
        ty2 = gt_ref[i, t, 3]
        iw = jnp.maximum(jnp.minimum(tx2, px2) - jnp.maximum(tx1, px1), 0.0)
        ih = jnp.maximum(jnp.minimum(ty2, py2) - jnp.maximum(ty1, py1), 0.0)
        inter = iw * ih
        area_t = (tx2 - tx1) * (ty2 - ty1)
        ov = inter / (area_t + area_p - inter)
        upd = ov > bo
        bo = jnp.where(upd, ov, bo)
        bi = jnp.where(upd, t, bi)
        m = jnp.max(ov)
        bp = jnp.min(jnp.where(ov == m, pidx, _PP))
        fidx = jnp.where(pidx == bp, t, fidx)
        return bo, bi, fidx

    _UNROLL = 10

    def match_body(it, carry):
        for u in range(_UNROLL):
            t = it * _UNROLL + u
            carry = tuple(match_one(i, t, *carry[i]) for i in range(_IMGS))
        return carry

    one_init = (
        jnp.full((_R, _L), -jnp.inf, f32),
        jnp.zeros((_R, _L), jnp.int32),
        jnp.full((_R, _L), -1, jnp.int32),
    )
    mres = jax.lax.fori_loop(0, _T // _UNROLL, match_body,
                             (one_init,) * _IMGS)

    bis = []
    for i in range(_IMGS):
        bo, bi, fidx = mres[i]
        forced = fidx >= 0
        bis.append((jnp.where(forced, 2.0, bo), jnp.where(forced, fidx, bi)))

    def gather_one(i, t, bi, lab, m1, m2, m3, m4):
        sel = bi == t
        lab = jnp.where(sel, gt_ref[i, t, 4], lab)
        m1 = jnp.where(sel, gt_ref[i, t, 0], m1)
        m2 = jnp.where(sel, gt_ref[i, t, 1], m2)
        m3 = jnp.where(sel, gt_ref[i, t, 2], m3)
        m4 = jnp.where(sel, gt_ref[i, t, 3], m4)
        return lab, m1, m2, m3, m4

    def gather_body(it, carry):
        for u in range(_UNROLL):
            t = it * _UNROLL + u
            carry = tuple(gather_one(i, t, bis[i][1], *carry[i])
                          for i in range(_IMGS))
        return carry

    z = jnp.zeros((_R, _L), f32)
    gres = jax.lax.fori_loop(0, _T // _UNROLL, gather_body,
                             ((z,) * 5,) * _IMGS)

    def sl1(x):
        ax = jnp.abs(x)
        return jnp.where(ax < 1.0, 0.5 * x * x, ax - 0.5)

    confs, loss_ls = [], []
    for i in range(_IMGS):
        bo, _ = bis[i]
        lab, mx1, my1, mx2, my2 = gres[i]
        conf = jnp.where(bo < _THRESH, 0.0, lab + 1.0)
        confs.append(conf)
        g0 = ((mx1 + mx2) / 2.0 - cx) / (0.1 * pw)
        g1 = ((my1 + my2) / 2.0 - cy) / (0.1 * ph)
        g2 = jnp.log((mx2 - mx1) / pw) / 0.2
        g3 = jnp.log((my2 - my1) / ph) / 0.2
        lsum = (sl1(loc_ref[i, 0] - g0) + sl1(loc_ref[i, 1] - g1)
                + sl1(loc_ref[i, 2] - g2) + sl1(loc_ref[i, 3] - g3))
        loss_ls.append(jnp.sum(jnp.where(conf > 0.0, lsum, 0.0)))

    ms = [score_ref[i, 0] for i in range(_IMGS)]
    for c in range(1, _C):
        ms = [jnp.maximum(ms[i], score_ref[i, c]) for i in range(_IMGS)]
    ss = [jnp.zeros((_R, _L), f32) for _ in range(_IMGS)]
    pks = [jnp.zeros((_R, _L), f32) for _ in range(_IMGS)]
    for c in range(_C):
        for i in range(_IMGS):
            x = score_ref[i, c]
            ss[i] = ss[i] + jnp.exp(x - ms[i])
            pks[i] = jnp.where(confs[i] == c, x, pks[i])

    ces, loss_cs, us, pos_ces, num_poss, ks = [], [], [], [], [], []
    for i in range(_IMGS):
        ce = ms[i] + jnp.log(ss[i]) - pks[i]
        pos = confs[i] > 0.0
        pos_ces.append(jnp.sum(jnp.where(pos, ce, 0.0)))
        num_pos = jnp.sum(pos.astype(jnp.int32))
        num_poss.append(num_pos)
        ks.append(jnp.minimum(_NEG_POS * num_pos, _P - 1))
        loss_c = jnp.where(pos, 0.0, ce)
        loss_c = jnp.where(valid, loss_c, 0.0)
        loss_cs.append(loss_c)
        us.append(jax.lax.bitcast_convert_type(loss_c, jnp.int32))

    def radix_body(r, accs):
        out = []
        for i in range(_IMGS):
            t_acc = accs[i]
            hi = jax.lax.shift_left(jnp.int32(1), 30 - 2 * r)
            lo = jax.lax.shift_left(jnp.int32(1), 29 - 2 * r)
            c1 = t_acc | lo
            c2 = t_acc | hi
            c3 = t_acc | hi | lo
            n1 = jnp.sum((us[i] >= c1).astype(jnp.int32))
            n2 = jnp.sum((us[i] >= c2).astype(jnp.int32))
            n3 = jnp.sum((us[i] >= c3).astype(jnp.int32))
            k = ks[i]
            t_acc = jnp.where(
                n3 >= k, c3,
                jnp.where(n2 >= k, c2, jnp.where(n1 >= k, c1, t_acc)))
            out.append(t_acc)
        return tuple(out)

    t_fins = jax.lax.fori_loop(0, 15, radix_body,
                               (jnp.int32(0),) * _IMGS)

    def last_body(accs):
        out = []
        for i in range(_IMGS):
            cand = accs[i] | 1
            cnt = jnp.sum((us[i] >= cand).astype(jnp.int32))
            out.append(jnp.where(cnt >= ks[i], cand, accs[i]))
        return tuple(out)

    t_fins = last_body(t_fins)

    for i in range(_IMGS):
        tau = jax.lax.bitcast_convert_type(t_fins[i], f32)
        loss_c = loss_cs[i]
        gt_mask = loss_c > tau
        sum_gt = jnp.sum(jnp.where(gt_mask, loss_c, 0.0))
        cnt_gt = jnp.sum(gt_mask.astype(jnp.int32))
        k = ks[i]
        topk = sum_gt + (k - cnt_gt).astype(f32) * tau
        topk = jnp.where(k > 0, topk, 0.0)
        o_l[i, :, :] = jnp.full((1, _L), loss_ls[i], f32)
        o_c[i, :, :] = jnp.full((1, _L), pos_ces[i], f32)
        o_k[i, :, :] = jnp.full((1, _L), topk, f32)
        o_n[i, :, :] = jnp.full((1, _L), num_poss[i].astype(f32), f32)


@jax.jit
def kernel(pred_loc, pred_score, priors, gt_data):
    f32 = jnp.float32
    pad = _PP - _P
    loc_t = jnp.pad(pred_loc.transpose(0, 2, 1), ((0, 0), (0, 0), (0, pad)))
    loc_t = loc_t.reshape(_B, 4, _R, _L)
    score_t = jnp.pad(pred_score.transpose(0, 2, 1),
                      ((0, 0), (0, 0), (0, pad)))
    score_t = score_t.reshape(_B, _C, _R, _L)
    pad_rows = jnp.tile(jnp.array([[-100.0, -100.0, 0.0, 0.0]], f32),
                        (pad, 1))
    pri_t = jnp.concatenate([priors, pad_rows], axis=0).T.reshape(4, _R, _L)

    grid = (_B // _IMGS,)
    out_shape = [jax.ShapeDtypeStruct((_B, 1, _L), f32)] * 4
    res = pl.pallas_call(
        _loss_kernel,
        grid=grid,
        in_specs=[
            pl.BlockSpec((_IMGS, _T, 5), lambda b: (b, 0, 0),
                         memory_space=pltpu.SMEM),
            pl.BlockSpec((_IMGS, 4, _R, _L), lambda b: (b, 0, 0, 0)),
            pl.BlockSpec((_IMGS, _C, _R, _L), lambda b: (b, 0, 0, 0)),
            pl.BlockSpec((4, _R, _L), lambda b: (0, 0, 0)),
        ],
        out_specs=[pl.BlockSpec((_IMGS, 1, _L), lambda b: (b, 0, 0))] * 4,
        out_shape=out_shape,
        compiler_params=pltpu.CompilerParams(
            dimension_semantics=("parallel",),
            vmem_limit_bytes=100 * 1024 * 1024),
    )(gt_data, loc_t, score_t, pri_t)
    l, c, k, n = (r[:, 0, 0] for r in res)
    n_tot = jnp.sum(n)
    return jnp.sum(l) / n_tot, (jnp.sum(c) + jnp.sum(k)) / n_tot
